# baseline (speedup 1.0000x reference)
import functools

import jax
import jax.numpy as jnp
from jax import lax
from jax.experimental import pallas as pl
from jax.experimental.pallas import tpu as pltpu
from jax.experimental.pallas import tpu_sc as plsc

EMB = 32
EPS = 1e-12
HALF = 16
NW = 32
CHUNK = 128
PITCH = 33


def kernel(entity_ids, table, gamma, beta):
    bsz, hist = entity_ids.shape
    nrows = bsz * hist
    rows_pw = nrows // NW
    nchunks = rows_pw // CHUNK
    bhi = bsz // CHUNK
    ids_flat = entity_ids.astype(jnp.int32).T.reshape(nrows)

    mesh = plsc.VectorSubcoreMesh(core_axis_name="c", subcore_axis_name="s")

    @functools.partial(
        pl.kernel,
        out_type=jax.ShapeDtypeStruct((hist, 4, bhi, 8 * CHUNK), jnp.float32),
        mesh=mesh,
        scratch_types=[
            pltpu.VMEM((rows_pw,), jnp.int32),
            pltpu.VMEM((4, CHUNK, EMB), jnp.float32),
            pltpu.VMEM((CHUNK * PITCH,), jnp.float32),
            pltpu.VMEM((2, 4, 8 * CHUNK), jnp.float32),
            pltpu.VMEM((EMB,), jnp.float32),
            pltpu.VMEM((EMB,), jnp.float32),
            pltpu.VMEM((EMB * HALF,), jnp.float32),
            pltpu.VMEM((EMB * HALF,), jnp.float32),
            pltpu.SemaphoreType.DMA,
            pltpu.SemaphoreType.DMA,
            pltpu.SemaphoreType.DMA,
            pltpu.SemaphoreType.DMA,
            pltpu.SemaphoreType.DMA,
            pltpu.SemaphoreType.DMA,
        ],
        compiler_params=pltpu.CompilerParams(
            needs_layout_passes=False, use_tc_tiling_on_sc=False),
    )
    def sc_kernel(ids_hbm, table_hbm, gamma_hbm, beta_hbm, out_hbm,
                  idx_v, data_v, pad_v, norm_v, gam_v, bet_v, gsp_v, bsp_v,
                  gsem0, gsem1, gsem2, gsem3, ssem0, ssem1):
        gsem = (gsem0, gsem1, gsem2, gsem3)
        ssem = (ssem0, ssem1)
        wid = lax.axis_index("s") * 2 + lax.axis_index("c")
        pltpu.sync_copy(ids_hbm.at[pl.ds(wid * rows_pw, rows_pw)], idx_v)
        pltpu.sync_copy(gamma_hbm, gam_v)
        pltpu.sync_copy(beta_hbm, bet_v)
        for half in range(2):
            gh = gam_v[pl.ds(half * HALF, HALF)]
            bh = bet_v[pl.ds(half * HALF, HALF)]
            for j in range(HALF):
                c = half * HALF + j
                gsp_v[pl.ds(c * HALF, HALF)] = jnp.full(
                    (HALF,), gh[j], jnp.float32)
                bsp_v[pl.ds(c * HALF, HALF)] = jnp.full(
                    (HALF,), bh[j], jnp.float32)
        iota_p = lax.iota(jnp.int32, HALF) * PITCH
        chunk0 = wid * nchunks

        def start_gather(k, slot):
            pltpu.async_copy(
                table_hbm.at[idx_v.at[pl.ds(k * CHUNK, CHUNK)]],
                data_v.at[slot], gsem[slot])

        def out_copies(k, nslot):
            g = chunk0 + k
            h = g // bhi
            b = g % bhi
            for ch in range(4):
                yield pltpu.make_async_copy(
                    norm_v.at[nslot, ch], out_hbm.at[h, ch, b], ssem[nslot])

        def process_chunk(k, slot, nslot):
            pltpu.make_async_copy(
                table_hbm.at[idx_v.at[pl.ds(k * CHUNK, CHUNK)]],
                data_v.at[slot], gsem[slot]).wait()

            @pl.when(k >= 2)
            def _():
                for cp in out_copies(k - 2, nslot):
                    cp.wait()

            def repack(r):
                pad_v[pl.ds(r * PITCH, HALF)] = \
                    data_v[slot, r, pl.ds(0, HALF)]
                pad_v[pl.ds(r * PITCH + HALF, HALF)] = \
                    data_v[slot, r, pl.ds(HALF, HALF)]

            plsc.parallel_loop(0, CHUNK, 1, unroll=8)(repack)

            def group(grp):
                col = iota_p + grp * (HALF * PITCH)
                acc_s = [jnp.zeros((HALF,), jnp.float32)] * 4
                acc_q = [jnp.zeros((HALF,), jnp.float32)] * 4
                for c in range(EMB):
                    v = plsc.load_gather(pad_v, [col + c])
                    acc_s[c % 4] = acc_s[c % 4] + v
                    acc_q[c % 4] = acc_q[c % 4] + v * v
                s = (acc_s[0] + acc_s[1]) + (acc_s[2] + acc_s[3])
                q2 = (acc_q[0] + acc_q[1]) + (acc_q[2] + acc_q[3])
                mean = s * (1.0 / EMB)
                var = jnp.maximum(q2 * (1.0 / EMB) - mean * mean, 0.0) + EPS
                i = lax.bitcast_convert_type(var, jnp.int32)
                i = jnp.int32(0x5F3759DF) - lax.shift_right_logical(i, 1)
                y = lax.bitcast_convert_type(i, jnp.float32)
                xh = var * 0.5
                y = y * (1.5 - xh * y * y)
                y = y * (1.5 - xh * y * y)
                y = y * (1.5 - xh * y * y)
                for c in range(EMB):
                    v = plsc.load_gather(pad_v, [col + c])
                    gsv = gsp_v[pl.ds(c * HALF, HALF)]
                    bsv = bsp_v[pl.ds(c * HALF, HALF)]
                    o = (v - mean) * (y * gsv) + bsv
                    norm_v[nslot, c // 8,
                           pl.ds((c % 8) * CHUNK + grp * HALF, HALF)] = o

            plsc.parallel_loop(0, CHUNK // HALF, 1)(group)

            for cp in out_copies(k, nslot):
                cp.start()

        for k in range(3):
            start_gather(k, k)

        def quad(p, _):
            for j in range(4):
                k = 4 * p + j
                process_chunk(k, j, j % 2)

                @pl.when(k + 3 < nchunks)
                def _():
                    start_gather(k + 3, (j + 3) % 4)
            return 0

        lax.fori_loop(0, nchunks // 4, quad, 0)

        for j in range(2):
            for cp in out_copies(nchunks - 2 + j, j):
                cp.wait()

    out4 = sc_kernel(ids_flat, table, gamma, beta)
    out5 = out4.reshape(hist, 4, bhi, 8, CHUNK)
    return out5.transpose(2, 4, 0, 1, 3).reshape(bsz, hist, EMB)
